# initial kernel scaffold (unmeasured)
import jax
import jax.numpy as jnp
from jax import lax
from jax.experimental import pallas as pl
from jax.experimental.pallas import tpu as pltpu

N_DEV = 32


def _gelu(y):
    c = 0.7978845608028654
    return 0.5 * y * (1.0 + jnp.tanh(c * (y + 0.044715 * y * y * y)))


def kernel(x, w_mat):
    m, _ = x.shape
    _, n = w_mat.shape
    chunk = m // N_DEV

    def body(x_ref, w_ref, out_ref, comm_ref, acc_ref,
             send_sems, recv_sems, credit_sem, out_sem):
        me = lax.axis_index("i")
        left = lax.rem(me + N_DEV - 1, N_DEV)
        right = lax.rem(me + 1, N_DEV)

        barrier = pltpu.get_barrier_semaphore()
        pl.semaphore_signal(barrier, inc=1, device_id=(left,),
                            device_id_type=pl.DeviceIdType.MESH)
        pl.semaphore_signal(barrier, inc=1, device_id=(right,),
                            device_id_type=pl.DeviceIdType.MESH)
        pl.semaphore_wait(barrier, 2)

        def partial_chunk(c):
            xs = x_ref[pl.ds(c * chunk, chunk), :]
            return jnp.dot(xs, w_ref[...], preferred_element_type=jnp.float32)

        comm_ref[0] = partial_chunk(me)

        n_steps = 2 * (N_DEV - 1)
        for s in range(n_steps):
            send_slot = s % 2
            recv_slot = (s + 1) % 2

            if s >= 1:
                pl.semaphore_wait(credit_sem, 1)

            rdma = pltpu.make_async_remote_copy(
                src_ref=comm_ref.at[send_slot],
                dst_ref=comm_ref.at[recv_slot],
                send_sem=send_sems.at[send_slot],
                recv_sem=recv_sems.at[recv_slot],
                device_id=(right,),
                device_id_type=pl.DeviceIdType.MESH,
            )
            rdma.start()

            if s <= N_DEV - 2:
                c_in = lax.rem(me - s - 1 + N_DEV, N_DEV)
                acc_ref[...] = partial_chunk(c_in)

            rdma.wait()
            if s < n_steps - 1:
                pl.semaphore_signal(credit_sem, inc=1, device_id=(left,),
                                    device_id_type=pl.DeviceIdType.MESH)

            if s < N_DEV - 2:
                comm_ref[recv_slot] = comm_ref[recv_slot] + acc_ref[...]
            elif s == N_DEV - 2:
                comm_ref[recv_slot] = _gelu(comm_ref[recv_slot] + acc_ref[...])
                own = lax.rem(me + 1, N_DEV)
                cp = pltpu.make_async_copy(
                    comm_ref.at[recv_slot],
                    out_ref.at[pl.ds(own * chunk, chunk), :],
                    out_sem,
                )
                cp.start()
                cp.wait()
            else:
                t = s - (N_DEV - 1)
                c_got = lax.rem(me - t + N_DEV, N_DEV)
                cp = pltpu.make_async_copy(
                    comm_ref.at[recv_slot],
                    out_ref.at[pl.ds(c_got * chunk, chunk), :],
                    out_sem,
                )
                cp.start()
                cp.wait()

    return pl.pallas_call(
        body,
        out_shape=jax.ShapeDtypeStruct((m, n), jnp.float32),
        in_specs=[
            pl.BlockSpec(memory_space=pltpu.VMEM),
            pl.BlockSpec(memory_space=pltpu.VMEM),
        ],
        out_specs=pl.BlockSpec(memory_space=pltpu.ANY),
        scratch_shapes=[
            pltpu.VMEM((2, chunk, n), jnp.float32),
            pltpu.VMEM((chunk, n), jnp.float32),
            pltpu.SemaphoreType.DMA((2,)),
            pltpu.SemaphoreType.DMA((2,)),
            pltpu.SemaphoreType.REGULAR,
            pltpu.SemaphoreType.DMA,
        ],
        compiler_params=pltpu.CompilerParams(collective_id=0),
    )(x, w_mat)


# baseline (device time: 3325809 ns/iter reference)
import jax
import jax.numpy as jnp
from jax import lax
from jax.experimental import pallas as pl
from jax.experimental.pallas import tpu as pltpu

N_DEV = 32


def _gelu(y):
    c = 0.7978845608028654
    return 0.5 * y * (1.0 + jnp.tanh(c * (y + 0.044715 * y * y * y)))


def kernel(x, w_mat):
    m, _ = x.shape
    _, n = w_mat.shape
    chunk = m // N_DEV

    def body(x_ref, w_ref, out_ref, comm_ref, acc_ref,
             send_sems, recv_sems, credit_sem, out_sem):
        me = lax.axis_index("i")
        left = lax.rem(me + N_DEV - 1, N_DEV)
        right = lax.rem(me + 1, N_DEV)

        barrier = pltpu.get_barrier_semaphore()
        pl.semaphore_signal(barrier, inc=1, device_id=(left,),
                            device_id_type=pl.DeviceIdType.MESH)
        pl.semaphore_signal(barrier, inc=1, device_id=(right,),
                            device_id_type=pl.DeviceIdType.MESH)
        pl.semaphore_wait(barrier, 2)

        def partial_chunk(c):
            xs = x_ref[pl.ds(c * chunk, chunk), :]
            return jnp.dot(xs, w_ref[...], preferred_element_type=jnp.float32)

        comm_ref[0] = partial_chunk(me)

        n_steps = 2 * (N_DEV - 1)
        for s in range(n_steps):
            send_slot = s % 2
            recv_slot = (s + 1) % 2

            if s >= 1:
                pl.semaphore_wait(credit_sem, 1)

            rdma = pltpu.make_async_remote_copy(
                src_ref=comm_ref.at[send_slot],
                dst_ref=comm_ref.at[recv_slot],
                send_sem=send_sems.at[send_slot],
                recv_sem=recv_sems.at[recv_slot],
                device_id=(right,),
                device_id_type=pl.DeviceIdType.MESH,
            )
            rdma.start()

            if s <= N_DEV - 2:
                c_in = lax.rem(me - s - 1 + N_DEV, N_DEV)
                acc_ref[...] = partial_chunk(c_in)

            rdma.wait()
            if s < n_steps - 1:
                pl.semaphore_signal(credit_sem, inc=1, device_id=(left,),
                                    device_id_type=pl.DeviceIdType.MESH)

            if s < N_DEV - 2:
                comm_ref[recv_slot] = comm_ref[recv_slot] + acc_ref[...]
            elif s == N_DEV - 2:
                comm_ref[recv_slot] = _gelu(comm_ref[recv_slot] + acc_ref[...])
                own = lax.rem(me + 1, N_DEV)
                cp = pltpu.make_async_copy(
                    comm_ref.at[recv_slot],
                    out_ref.at[pl.ds(own * chunk, chunk), :],
                    out_sem,
                )
                cp.start()
                cp.wait()
            else:
                t = s - (N_DEV - 1)
                c_got = lax.rem(me - t + N_DEV, N_DEV)
                cp = pltpu.make_async_copy(
                    comm_ref.at[recv_slot],
                    out_ref.at[pl.ds(c_got * chunk, chunk), :],
                    out_sem,
                )
                cp.start()
                cp.wait()

    return pl.pallas_call(
        body,
        out_shape=jax.ShapeDtypeStruct((m, n), jnp.float32),
        in_specs=[
            pl.BlockSpec(memory_space=pltpu.VMEM),
            pl.BlockSpec(memory_space=pltpu.VMEM),
        ],
        out_specs=pl.BlockSpec(memory_space=pl.ANY),
        scratch_shapes=[
            pltpu.VMEM((2, chunk, n), jnp.float32),
            pltpu.VMEM((chunk, n), jnp.float32),
            pltpu.SemaphoreType.DMA((2,)),
            pltpu.SemaphoreType.DMA((2,)),
            pltpu.SemaphoreType.REGULAR,
            pltpu.SemaphoreType.DMA,
        ],
        compiler_params=pltpu.CompilerParams(collective_id=0),
    )(x, w_mat)


# device time: 2887961 ns/iter; 1.1516x vs baseline; 1.1516x over previous
import jax
import jax.numpy as jnp
from jax import lax
from jax.experimental import pallas as pl
from jax.experimental.pallas import tpu as pltpu

N_DEV = 32


def _gelu(y):
    c = 0.7978845608028654
    return 0.5 * y * (1.0 + jnp.tanh(c * (y + 0.044715 * y * y * y)))


def kernel(x, w_mat):
    m, _ = x.shape
    _, n = w_mat.shape
    chunk = m // N_DEV
    q = n // 4

    n_steps = 2 * (N_DEV - 1)

    def body(x_ref, w_ref, out_ref,
             comm0, comm1, comm2, comm3,
             acc0, acc1, acc2, acc3,
             send_sems, recv_sems, credit_sems, out_sem):
        me = lax.axis_index("i")
        left = lax.rem(me + N_DEV - 1, N_DEV)
        right = lax.rem(me + 1, N_DEV)

        rings = [
            (0, comm0, acc0, 0 * q, True),
            (2, comm2, acc2, 2 * q, False),
            (1, comm1, acc1, 1 * q, True),
            (3, comm3, acc3, 3 * q, False),
        ]

        def dst_dev(fwd):
            return (right,) if fwd else (left,)

        def upstream(fwd):
            return (left,) if fwd else (right,)

        def arrival_chunk(s, fwd):
            rs = lax.rem(me - s - 1 + 3 * N_DEV, N_DEV)
            ag = lax.rem(me - (s - (N_DEV - 1)) + 3 * N_DEV, N_DEV)
            if not fwd:
                rs = lax.rem(me + s + 1, N_DEV)
                ag = lax.rem(me + s - (N_DEV - 1) + 3 * N_DEV, N_DEV)
            return jnp.where(s < N_DEV - 1, rs, ag)

        def partial(c, col0):
            xs = x_ref[pl.ds(c * chunk, chunk), :]
            return jnp.dot(xs, w_ref[:, col0:col0 + q],
                           preferred_element_type=jnp.float32)

        def desc(r, comm, fwd, slot_send, slot_recv):
            return pltpu.make_async_remote_copy(
                src_ref=comm.at[slot_send],
                dst_ref=comm.at[slot_recv],
                send_sem=send_sems.at[r, slot_send],
                recv_sem=recv_sems.at[r, slot_recv],
                device_id=dst_dev(fwd),
                device_id_type=pl.DeviceIdType.MESH,
            )

        barrier = pltpu.get_barrier_semaphore()
        pl.semaphore_signal(barrier, inc=1, device_id=(left,),
                            device_id_type=pl.DeviceIdType.MESH)
        pl.semaphore_signal(barrier, inc=1, device_id=(right,),
                            device_id_type=pl.DeviceIdType.MESH)
        pl.semaphore_wait(barrier, 2)

        for r, comm, acc, col0, fwd in rings:
            comm[0] = partial(me, col0)
            desc(r, comm, fwd, 0, 1).start()
        for r, comm, acc, col0, fwd in rings:
            acc[...] = partial(arrival_chunk(jnp.int32(0), fwd), col0)

        def recv_and_credit(s, r, comm, fwd, slot_send, slot_recv,
                            last=False):
            d = desc(r, comm, fwd, slot_send, slot_recv)
            d.wait_recv()
            d.wait_send()
            if not last:
                pl.semaphore_signal(credit_sems.at[r], inc=1,
                                    device_id=upstream(fwd),
                                    device_id_type=pl.DeviceIdType.MESH)

        def start_next(s, r, comm, fwd, slot_send, slot_recv):
            pl.semaphore_wait(credit_sems.at[r], 1)
            desc(r, comm, fwd, slot_recv, slot_send).start()

        def store_out(comm, slot, c, col0):
            cp = pltpu.make_async_copy(
                comm.at[slot],
                out_ref.at[pl.ds(c * chunk, chunk), pl.ds(col0, q)],
                out_sem,
            )
            cp.start()
            cp.wait()

        def rs_step(s, carry):
            slot_send = lax.rem(s, 2)
            slot_recv = 1 - slot_send
            for r, comm, acc, col0, fwd in rings:
                recv_and_credit(s, r, comm, fwd, slot_send, slot_recv)
                comm[slot_recv] = comm[slot_recv] + acc[...]
                start_next(s, r, comm, fwd, slot_send, slot_recv)
                acc[...] = partial(arrival_chunk(s + 1, fwd), col0)
            return carry

        lax.fori_loop(0, N_DEV - 2, rs_step, 0, unroll=False)

        s30 = jnp.int32(N_DEV - 2)
        slot_send = (N_DEV - 2) % 2
        slot_recv = 1 - slot_send
        for r, comm, acc, col0, fwd in rings:
            recv_and_credit(s30, r, comm, fwd, slot_send, slot_recv)
            comm[slot_recv] = _gelu(comm[slot_recv] + acc[...])
            own = arrival_chunk(s30, fwd)
            store_out(comm, slot_recv, own, col0)
            start_next(s30, r, comm, fwd, slot_send, slot_recv)

        def ag_step(s, carry):
            slot_send = lax.rem(s, 2)
            slot_recv = 1 - slot_send
            for r, comm, acc, col0, fwd in rings:
                recv_and_credit(s, r, comm, fwd, slot_send, slot_recv)
                store_out(comm, slot_recv, arrival_chunk(s, fwd), col0)
                start_next(s, r, comm, fwd, slot_send, slot_recv)
            return carry

        lax.fori_loop(N_DEV - 1, n_steps - 1, ag_step, 0, unroll=False)

        s_last = jnp.int32(n_steps - 1)
        slot_send = (n_steps - 1) % 2
        slot_recv = 1 - slot_send
        for r, comm, acc, col0, fwd in rings:
            recv_and_credit(s_last, r, comm, fwd, slot_send, slot_recv,
                            last=True)
            store_out(comm, slot_recv, arrival_chunk(s_last, fwd), col0)

    return pl.pallas_call(
        body,
        out_shape=jax.ShapeDtypeStruct((m, n), jnp.float32),
        in_specs=[
            pl.BlockSpec(memory_space=pltpu.VMEM),
            pl.BlockSpec(memory_space=pltpu.VMEM),
        ],
        out_specs=pl.BlockSpec(memory_space=pl.ANY),
        scratch_shapes=[
            pltpu.VMEM((2, chunk, q), jnp.float32),
            pltpu.VMEM((2, chunk, q), jnp.float32),
            pltpu.VMEM((2, chunk, q), jnp.float32),
            pltpu.VMEM((2, chunk, q), jnp.float32),
            pltpu.VMEM((chunk, q), jnp.float32),
            pltpu.VMEM((chunk, q), jnp.float32),
            pltpu.VMEM((chunk, q), jnp.float32),
            pltpu.VMEM((chunk, q), jnp.float32),
            pltpu.SemaphoreType.DMA((4, 2)),
            pltpu.SemaphoreType.DMA((4, 2)),
            pltpu.SemaphoreType.REGULAR((4,)),
            pltpu.SemaphoreType.DMA,
        ],
        compiler_params=pltpu.CompilerParams(collective_id=0),
    )(x, w_mat)


# device time: 1494853 ns/iter; 2.2248x vs baseline; 1.9319x over previous
import jax
import jax.numpy as jnp
from jax import lax
from jax.experimental import pallas as pl
from jax.experimental.pallas import tpu as pltpu

N_DEV = 32


def _gelu(y):
    c = 0.7978845608028654
    return 0.5 * y * (1.0 + jnp.tanh(c * (y + 0.044715 * y * y * y)))


def kernel(x, w_mat):
    m, _ = x.shape
    _, n = w_mat.shape
    chunk = m // N_DEV
    q = n // 4

    n_steps = 2 * (N_DEV - 1)

    def body(x_ref, w_ref, out_ref,
             comm0, comm1, comm2, comm3,
             acc0, acc1, acc2, acc3,
             send_sems, recv_sems, credit_sems, out_sem):
        def k_of_pos(p):
            z = p // 8
            w = lax.rem(p, 8)
            y = w // 2
            xb = lax.rem(w, 2)
            ye = lax.rem(y, 2)
            xc = jnp.where(ye == 0, xb, 1 - xb)
            idx0 = y * 4 + jnp.where(ye == 0, z, 3 - z)
            return jnp.where(xc == 0, idx0, N_DEV - 1 - idx0)

        def pos_of_k(k):
            k = lax.rem(k + 2 * N_DEV, N_DEV)
            xc = jnp.where(k < 16, 0, 1)
            idx0 = jnp.where(k < 16, k, N_DEV - 1 - k)
            y = idx0 // 4
            zz = lax.rem(idx0, 4)
            ye = lax.rem(y, 2)
            z = jnp.where(ye == 0, zz, 3 - zz)
            xb = jnp.where(ye == 0, xc, 1 - xc)
            return z * 8 + y * 2 + xb

        me = k_of_pos(lax.axis_index("i"))
        left = pos_of_k(me + N_DEV - 1)
        right = pos_of_k(me + 1)

        rings = [
            (0, comm0, acc0, 0 * q, True),
            (2, comm2, acc2, 2 * q, False),
            (1, comm1, acc1, 1 * q, True),
            (3, comm3, acc3, 3 * q, False),
        ]

        def dst_dev(fwd):
            return (right,) if fwd else (left,)

        def upstream(fwd):
            return (left,) if fwd else (right,)

        def arrival_chunk(s, fwd):
            rs = lax.rem(me - s - 1 + 3 * N_DEV, N_DEV)
            ag = lax.rem(me - (s - (N_DEV - 1)) + 3 * N_DEV, N_DEV)
            if not fwd:
                rs = lax.rem(me + s + 1, N_DEV)
                ag = lax.rem(me + s - (N_DEV - 1) + 3 * N_DEV, N_DEV)
            return jnp.where(s < N_DEV - 1, rs, ag)

        def partial(c, col0):
            xs = x_ref[pl.ds(pos_of_k(c) * chunk, chunk), :]
            return jnp.dot(xs, w_ref[:, col0:col0 + q],
                           preferred_element_type=jnp.float32)

        def desc(r, comm, fwd, slot_send, slot_recv):
            return pltpu.make_async_remote_copy(
                src_ref=comm.at[slot_send],
                dst_ref=comm.at[slot_recv],
                send_sem=send_sems.at[r, slot_send],
                recv_sem=recv_sems.at[r, slot_recv],
                device_id=dst_dev(fwd),
                device_id_type=pl.DeviceIdType.MESH,
            )

        barrier = pltpu.get_barrier_semaphore()
        pl.semaphore_signal(barrier, inc=1, device_id=(left,),
                            device_id_type=pl.DeviceIdType.MESH)
        pl.semaphore_signal(barrier, inc=1, device_id=(right,),
                            device_id_type=pl.DeviceIdType.MESH)
        pl.semaphore_wait(barrier, 2)

        for r, comm, acc, col0, fwd in rings:
            comm[0] = partial(me, col0)
            desc(r, comm, fwd, 0, 1).start()
        for r, comm, acc, col0, fwd in rings:
            acc[...] = partial(arrival_chunk(jnp.int32(0), fwd), col0)

        def recv_and_credit(s, r, comm, fwd, slot_send, slot_recv,
                            last=False):
            d = desc(r, comm, fwd, slot_send, slot_recv)
            d.wait_recv()
            d.wait_send()
            if not last:
                pl.semaphore_signal(credit_sems.at[r], inc=1,
                                    device_id=upstream(fwd),
                                    device_id_type=pl.DeviceIdType.MESH)

        def start_next(s, r, comm, fwd, slot_send, slot_recv):
            pl.semaphore_wait(credit_sems.at[r], 1)
            desc(r, comm, fwd, slot_recv, slot_send).start()

        def store_out(comm, slot, c, col0):
            cp = pltpu.make_async_copy(
                comm.at[slot],
                out_ref.at[pl.ds(pos_of_k(c) * chunk, chunk),
                           pl.ds(col0, q)],
                out_sem,
            )
            cp.start()
            cp.wait()

        def rs_step(s, carry):
            slot_send = lax.rem(s, 2)
            slot_recv = 1 - slot_send
            for r, comm, acc, col0, fwd in rings:
                recv_and_credit(s, r, comm, fwd, slot_send, slot_recv)
                comm[slot_recv] = comm[slot_recv] + acc[...]
                start_next(s, r, comm, fwd, slot_send, slot_recv)
                acc[...] = partial(arrival_chunk(s + 1, fwd), col0)
            return carry

        lax.fori_loop(0, N_DEV - 2, rs_step, 0, unroll=False)

        s30 = jnp.int32(N_DEV - 2)
        slot_send = (N_DEV - 2) % 2
        slot_recv = 1 - slot_send
        for r, comm, acc, col0, fwd in rings:
            recv_and_credit(s30, r, comm, fwd, slot_send, slot_recv)
            comm[slot_recv] = _gelu(comm[slot_recv] + acc[...])
            own = arrival_chunk(s30, fwd)
            store_out(comm, slot_recv, own, col0)
            start_next(s30, r, comm, fwd, slot_send, slot_recv)

        def ag_step(s, carry):
            slot_send = lax.rem(s, 2)
            slot_recv = 1 - slot_send
            for r, comm, acc, col0, fwd in rings:
                recv_and_credit(s, r, comm, fwd, slot_send, slot_recv)
                store_out(comm, slot_recv, arrival_chunk(s, fwd), col0)
                start_next(s, r, comm, fwd, slot_send, slot_recv)
            return carry

        lax.fori_loop(N_DEV - 1, n_steps - 1, ag_step, 0, unroll=False)

        s_last = jnp.int32(n_steps - 1)
        slot_send = (n_steps - 1) % 2
        slot_recv = 1 - slot_send
        for r, comm, acc, col0, fwd in rings:
            recv_and_credit(s_last, r, comm, fwd, slot_send, slot_recv,
                            last=True)
            store_out(comm, slot_recv, arrival_chunk(s_last, fwd), col0)

    return pl.pallas_call(
        body,
        out_shape=jax.ShapeDtypeStruct((m, n), jnp.float32),
        in_specs=[
            pl.BlockSpec(memory_space=pltpu.VMEM),
            pl.BlockSpec(memory_space=pltpu.VMEM),
        ],
        out_specs=pl.BlockSpec(memory_space=pl.ANY),
        scratch_shapes=[
            pltpu.VMEM((2, chunk, q), jnp.float32),
            pltpu.VMEM((2, chunk, q), jnp.float32),
            pltpu.VMEM((2, chunk, q), jnp.float32),
            pltpu.VMEM((2, chunk, q), jnp.float32),
            pltpu.VMEM((chunk, q), jnp.float32),
            pltpu.VMEM((chunk, q), jnp.float32),
            pltpu.VMEM((chunk, q), jnp.float32),
            pltpu.VMEM((chunk, q), jnp.float32),
            pltpu.SemaphoreType.DMA((4, 2)),
            pltpu.SemaphoreType.DMA((4, 2)),
            pltpu.SemaphoreType.REGULAR((4,)),
            pltpu.SemaphoreType.DMA,
        ],
        compiler_params=pltpu.CompilerParams(collective_id=0),
    )(x, w_mat)
